# baseline (speedup 1.0000x reference)
import functools

import jax
import jax.numpy as jnp
from jax import lax
from jax.experimental import pallas as pl
from jax.experimental.pallas import tpu as pltpu
from jax.experimental.pallas import tpu_sc as plsc

_NE = 512
_D = 64
_B = 64
_N = 1024
_NTOK = _B * _N
_RB = 8
_IB = 8


def _vq_row(zt_ref, wt_ref, idx_ref, perp_ref, uniq_ref,
            sww_ref, wtm2_ref, counts_ref):
    i = pl.program_id(0)
    nb = pl.num_programs(0)

    @pl.when(i == 0)
    def _prep():
        wt = wt_ref[...]
        wtm2_ref[...] = wt * (-2.0)
        sww_ref[...] = jax.lax.dot_general(
            wt * wt, jnp.ones((_D, 1), jnp.float32),
            (((0,), (0,)), ((), ())))
        counts_ref[...] = jnp.zeros_like(counts_ref)

    cacc = None
    for j in range(_RB):
        zt = zt_ref[...].reshape(_RB, _D, _N)[j]

        s = zt * zt
        for k in (32, 16, 8, 4, 2, 1):
            s = jax.lax.slice(s, (0, 0), (k, _N)) + jax.lax.slice(
                s, (k, 0), (2 * k, _N))
        szz = s

        scores2 = jax.lax.dot_general(wtm2_ref[...], zt,
                                      (((0,), (0,)), ((), ())))
        dist = (szz + sww_ref[...]) + scores2

        dmin = jnp.min(dist, axis=0, keepdims=True)
        code = jax.lax.broadcasted_iota(jnp.int32, dist.shape, 0)
        idx = jnp.min(jnp.where(dist == dmin, code, _NE),
                      axis=0, keepdims=True)

        onehot = (code == idx).astype(jnp.float32)
        idx_ref[pl.ds(jax.lax.rem(i, _IB // _RB) * _RB + j, 1), :] = idx

        c = jax.lax.dot_general(
            onehot, jnp.ones((_N, 1), jnp.float32), (((1,), (0,)), ((), ())),
            preferred_element_type=jnp.float32)
        cacc = c if cacc is None else cacc + c
    counts_ref[...] += cacc

    @pl.when(i == nb - 1)
    def _fin():
        counts = counts_ref[...]
        avg = counts * (1.0 / _NTOK)
        ent = jnp.sum(avg * jnp.log(avg + 1e-10), axis=(0, 1), keepdims=True)
        perp_ref[...] = jnp.exp(-ent)
        uniq_ref[...] = jnp.sum((counts > 0.0).astype(jnp.int32),
                                axis=(0, 1), keepdims=True)


_SC_INFO = plsc.get_sparse_core_info()
_NW = _SC_INFO.num_cores * _SC_INFO.num_subcores
_BPW = _NTOK // _NW
_CHUNK = 512
_NCH = _BPW // _CHUNK


def _sc_gather(table_hbm, idx_hbm, out_hbm, idx_v, rows_v, sem):
    wid = lax.axis_index("s") * _SC_INFO.num_cores + lax.axis_index("c")
    base = wid * _BPW
    pltpu.sync_copy(idx_hbm.at[pl.ds(base, _BPW)], idx_v)
    for c in range(_NCH):
        pltpu.async_copy(
            table_hbm.at[idx_v.at[pl.ds(c * _CHUNK, _CHUNK)]],
            rows_v, sem).wait()
        pltpu.sync_copy(rows_v,
                        out_hbm.at[pl.ds(base + c * _CHUNK, _CHUNK)])


def kernel(z, W):
    original_dtype = z.dtype
    zt = jnp.swapaxes(z.astype(jnp.float32), 1, 2)
    wt = jnp.swapaxes(W, 0, 1)
    idx, perp, uniq = pl.pallas_call(
        _vq_row,
        grid=(_B // _RB,),
        in_specs=[
            pl.BlockSpec((_RB, _D, _N), lambda i: (i, 0, 0)),
            pl.BlockSpec((_D, _NE), lambda i: (0, 0)),
        ],
        out_specs=[
            pl.BlockSpec((_IB, _N), lambda i: (i // (_IB // _RB), 0)),
            pl.BlockSpec((1, 1), lambda i: (0, 0)),
            pl.BlockSpec((1, 1), lambda i: (0, 0)),
        ],
        out_shape=[
            jax.ShapeDtypeStruct((_B, _N), jnp.int32),
            jax.ShapeDtypeStruct((1, 1), jnp.float32),
            jax.ShapeDtypeStruct((1, 1), jnp.int32),
        ],
        scratch_shapes=[
            pltpu.VMEM((_NE, 1), jnp.float32),
            pltpu.VMEM((_D, _NE), jnp.float32),
            pltpu.VMEM((_NE, 1), jnp.float32),
        ],
        compiler_params=pltpu.CompilerParams(
            dimension_semantics=("arbitrary",)),
    )(zt, wt)

    gather = functools.partial(
        pl.kernel,
        mesh=plsc.VectorSubcoreMesh(core_axis_name="c", subcore_axis_name="s"),
        out_type=jax.ShapeDtypeStruct((_NTOK, 128), jnp.float32),
        scratch_types=[
            pltpu.VMEM((_BPW,), jnp.int32),
            pltpu.VMEM((_CHUNK, 128), jnp.float32),
            pltpu.SemaphoreType.DMA,
        ],
    )(_sc_gather)
    wp = jnp.pad(W, ((0, 0), (0, 128 - _D)))
    qwide = gather(wp, idx.reshape(_NTOK))
    quantized = qwide[:, :_D].reshape(_B, _N, _D).astype(original_dtype)
    vq_loss = jnp.zeros((), jnp.float32)
    return (quantized, idx, vq_loss, perp.reshape(()), uniq.reshape(()))


# speedup vs baseline: 1.9100x; 1.9100x over previous
import jax
import jax.numpy as jnp
from jax.experimental import pallas as pl
from jax.experimental.pallas import tpu as pltpu

_NE = 512
_D = 64
_B = 64
_N = 1024
_NTOK = _B * _N
_RB = 8
_IB = 8


def _vq_row(zt_ref, wt_ref, qt_ref, idx_ref, perp_ref, uniq_ref,
            sww_ref, wtm2_ref, counts_ref):
    i = pl.program_id(0)
    nb = pl.num_programs(0)

    @pl.when(i == 0)
    def _prep():
        wt = wt_ref[...]
        wtm2_ref[...] = wt * (-2.0)
        sww_ref[...] = jax.lax.dot_general(
            wt * wt, jnp.ones((_D, 1), jnp.float32),
            (((0,), (0,)), ((), ())))
        counts_ref[...] = jnp.zeros_like(counts_ref)

    cacc = None
    for j in range(_RB):
        zt = zt_ref[...].reshape(_RB, _D, _N)[j]

        s = zt * zt
        for k in (32, 16, 8, 4, 2, 1):
            s = jax.lax.slice(s, (0, 0), (k, _N)) + jax.lax.slice(
                s, (k, 0), (2 * k, _N))
        szz = s

        scores2 = jax.lax.dot_general(wtm2_ref[...], zt,
                                      (((0,), (0,)), ((), ())))
        dist = (szz + sww_ref[...]) + scores2

        dmin = jnp.min(dist, axis=0, keepdims=True)
        code = jax.lax.broadcasted_iota(jnp.int32, dist.shape, 0)
        idx = jnp.min(jnp.where(dist == dmin, code, _NE),
                      axis=0, keepdims=True)

        onehot = (code == idx).astype(jnp.float32)
        qt = jax.lax.dot_general(wt_ref[...], onehot, (((1,), (0,)), ((), ())),
                                 preferred_element_type=jnp.float32)
        qt_ref[pl.ds(j, 1), :, :] = (zt + (qt - zt)).reshape(1, _D, _N)
        idx_ref[pl.ds(jax.lax.rem(i, _IB // _RB) * _RB + j, 1), :] = idx

        c = jax.lax.dot_general(
            onehot, jnp.ones((_N, 1), jnp.float32), (((1,), (0,)), ((), ())),
            preferred_element_type=jnp.float32)
        cacc = c if cacc is None else cacc + c
    counts_ref[...] += cacc

    @pl.when(i == nb - 1)
    def _fin():
        counts = counts_ref[...]
        avg = counts * (1.0 / _NTOK)
        ent = jnp.sum(avg * jnp.log(avg + 1e-10), axis=(0, 1), keepdims=True)
        perp_ref[...] = jnp.exp(-ent)
        uniq_ref[...] = jnp.sum((counts > 0.0).astype(jnp.int32),
                                axis=(0, 1), keepdims=True)


def kernel(z, W):
    original_dtype = z.dtype
    zt = jnp.swapaxes(z.astype(jnp.float32), 1, 2)
    wt = jnp.swapaxes(W, 0, 1)
    qt, idx, perp, uniq = pl.pallas_call(
        _vq_row,
        grid=(_B // _RB,),
        in_specs=[
            pl.BlockSpec((_RB, _D, _N), lambda i: (i, 0, 0)),
            pl.BlockSpec((_D, _NE), lambda i: (0, 0)),
        ],
        out_specs=[
            pl.BlockSpec((_RB, _D, _N), lambda i: (i, 0, 0)),
            pl.BlockSpec((_IB, _N), lambda i: (i // (_IB // _RB), 0)),
            pl.BlockSpec((1, 1), lambda i: (0, 0)),
            pl.BlockSpec((1, 1), lambda i: (0, 0)),
        ],
        out_shape=[
            jax.ShapeDtypeStruct((_B, _D, _N), jnp.float32),
            jax.ShapeDtypeStruct((_B, _N), jnp.int32),
            jax.ShapeDtypeStruct((1, 1), jnp.float32),
            jax.ShapeDtypeStruct((1, 1), jnp.int32),
        ],
        scratch_shapes=[
            pltpu.VMEM((_NE, 1), jnp.float32),
            pltpu.VMEM((_D, _NE), jnp.float32),
            pltpu.VMEM((_NE, 1), jnp.float32),
        ],
        compiler_params=pltpu.CompilerParams(
            dimension_semantics=("arbitrary",)),
    )(zt, wt)
    quantized = jnp.swapaxes(qt, 1, 2).astype(original_dtype)
    vq_loss = jnp.zeros((), jnp.float32)
    return (quantized, idx, vq_loss, perp.reshape(()), uniq.reshape(()))
